# baseline (speedup 1.0000x reference)
import functools

import jax
import jax.numpy as jnp
from jax.experimental import pallas as pl
from jax.experimental.pallas import tpu as pltpu

_N = 4096
_TILE = 512
_K = _N // 2
_WEIGHT = 3.0


def _topk_stats(x, n, k):
    total = jnp.sum(x)
    mean_all = total / n
    xi = jax.lax.bitcast_convert_type(x, jnp.int32)

    def bit_body(b, kth):
        cand = kth | (jnp.int32(1) << (30 - b))
        cnt = jnp.sum(jnp.where(xi >= cand, jnp.int32(1), jnp.int32(0)))
        return jnp.where(cnt >= k, cand, kth)

    kth = jax.lax.fori_loop(0, 31, bit_body, jnp.int32(0))
    thr = jax.lax.bitcast_convert_type(kth, jnp.float32)
    gt_mask = xi > kth
    cnt_gt = jnp.sum(jnp.where(gt_mask, jnp.int32(1), jnp.int32(0)))
    sum_top = jnp.sum(jnp.where(gt_mask, x, 0.0)) + (k - cnt_gt).astype(jnp.float32) * thr
    return mean_all + _WEIGHT * sum_top / k


def _bf16(x):
    return x.astype(jnp.bfloat16).astype(jnp.float32)


def _chamfer_body(predT_ref, gt_ref, out_ref, rmB_ref):
    predT = predT_ref[0]
    px = predT[0:1, :]
    py = predT[1:2, :]
    pz = predT[2:3, :]
    p2row = px * px + py * py + pz * pz
    pxb, pyb, pzb = _bf16(px), _bf16(py), _bf16(pz)

    def tile_body(t, rmA):
        base = t * _TILE
        gx = gt_ref[0, pl.ds(base, _TILE), 0:1]
        gy = gt_ref[0, pl.ds(base, _TILE), 1:2]
        gz = gt_ref[0, pl.ds(base, _TILE), 2:3]
        g2 = gx * gx + gy * gy + gz * gz
        cross = _bf16(gx) * pxb + _bf16(gy) * pyb + _bf16(gz) * pzb
        e = g2 - 2.0 * cross
        rmA = jnp.minimum(rmA, jnp.min(e, axis=0, keepdims=True))
        f = e + p2row
        rmB_ref[pl.ds(base, _TILE), :] = jnp.min(f, axis=1, keepdims=True)
        return rmA

    rmA = jax.lax.fori_loop(
        0, _N // _TILE, tile_body, jnp.full((1, _N), jnp.inf, jnp.float32)
    )
    d1 = jnp.sqrt(jnp.maximum(rmA + p2row, 0.0))
    d2 = jnp.sqrt(jnp.maximum(rmB_ref[...], 0.0))
    loss = _topk_stats(d1, _N, _K) + _topk_stats(d2, _N, _K)
    out_ref[0] = jnp.full((8, 128), loss, jnp.float32)


@jax.jit
def kernel(pred_pointclouds, gt_pointclouds):
    b = pred_pointclouds.shape[0]
    predT = jnp.transpose(pred_pointclouds, (0, 2, 1))
    out = pl.pallas_call(
        _chamfer_body,
        grid=(b,),
        in_specs=[
            pl.BlockSpec((1, 3, _N), lambda i: (i, 0, 0)),
            pl.BlockSpec((1, _N, 3), lambda i: (i, 0, 0)),
        ],
        out_specs=pl.BlockSpec((1, 8, 128), lambda i: (i, 0, 0)),
        out_shape=jax.ShapeDtypeStruct((b, 8, 128), jnp.float32),
        scratch_shapes=[pltpu.VMEM((_N, 1), jnp.float32)],
    )(predT, gt_pointclouds)
    return jnp.sum(out[:, 0, 0]) / b
